# baseline (speedup 1.0000x reference)
import functools

import jax
import jax.numpy as jnp
from jax import lax
from jax.experimental import pallas as pl
from jax.experimental.pallas import tpu as pltpu
from jax.experimental.pallas import tpu_sc as plsc

_NUM_CORES = 2
_NUM_SUBCORES = 16
_NW = _NUM_CORES * _NUM_SUBCORES
_CHUNK = 128
_LANES = 16


def kernel(srcs, dsts, table):
    B = srcs.shape[0]
    D = table.shape[1]
    rows_per_w = B // _NW
    n_chunks = rows_per_w // _CHUNK

    srcs2 = srcs.reshape(B // _CHUNK, _CHUNK)
    dsts2 = dsts.reshape(B // _CHUNK, _CHUNK)

    mesh = plsc.VectorSubcoreMesh(
        core_axis_name="c", subcore_axis_name="s",
        num_cores=_NUM_CORES, num_subcores=_NUM_SUBCORES)

    @functools.partial(
        pl.kernel,
        out_type=(jax.ShapeDtypeStruct((B, D), jnp.float32),
                  jax.ShapeDtypeStruct((B, D), jnp.float32)),
        mesh=mesh,
        scratch_types=[
            pltpu.VMEM((n_chunks, _CHUNK), jnp.int32),
            pltpu.VMEM((n_chunks, _CHUNK), jnp.int32),
            pltpu.SemaphoreType.DMA,
        ],
        compiler_params=pltpu.CompilerParams(needs_layout_passes=False),
    )
    def deepwalk_lookup(srcs_hbm, dsts_hbm, table_hbm, out_s, out_d,
                        idx_s, idx_d, sem):
        wid = lax.axis_index("s") * _NUM_CORES + lax.axis_index("c")
        crow = wid * n_chunks
        base = wid * rows_per_w
        pltpu.sync_copy(srcs_hbm.at[pl.ds(crow, n_chunks)], idx_s)
        pltpu.sync_copy(dsts_hbm.at[pl.ds(crow, n_chunks)], idx_d)

        def run_list(idx_ref, out_ref):
            def chunk_body(c, carry):
                copies = []
                for r in range(_CHUNK // _LANES):
                    v = idx_ref[c, pl.ds(r * _LANES, _LANES)]
                    for l in range(_LANES):
                        i = v[l]
                        copies.append(pltpu.async_copy(
                            table_hbm.at[i],
                            out_ref.at[base + c * _CHUNK + r * _LANES + l],
                            sem))
                for cp in copies:
                    cp.wait()
                return carry

            lax.fori_loop(0, n_chunks, chunk_body, 0)

        run_list(idx_s, out_s)
        run_list(idx_d, out_d)

    return deepwalk_lookup(srcs2, dsts2, table)


# speedup vs baseline: 1.3487x; 1.3487x over previous
import functools

import jax
import jax.numpy as jnp
from jax import lax
from jax.experimental import pallas as pl
from jax.experimental.pallas import tpu as pltpu
from jax.experimental.pallas import tpu_sc as plsc

_NUM_CORES = 2
_NUM_SUBCORES = 16
_NW = _NUM_CORES * _NUM_SUBCORES
_CHUNK = 128


def kernel(srcs, dsts, table):
    B = srcs.shape[0]
    D = table.shape[1]
    rows_per_w = B // _NW
    n_chunks = rows_per_w // _CHUNK

    srcs2 = srcs.reshape(B // _CHUNK, _CHUNK)
    dsts2 = dsts.reshape(B // _CHUNK, _CHUNK)

    mesh = plsc.VectorSubcoreMesh(
        core_axis_name="c", subcore_axis_name="s",
        num_cores=_NUM_CORES, num_subcores=_NUM_SUBCORES)

    @functools.partial(
        pl.kernel,
        out_type=(jax.ShapeDtypeStruct((B, D), jnp.float32),
                  jax.ShapeDtypeStruct((B, D), jnp.float32)),
        mesh=mesh,
        scratch_types=[
            pltpu.VMEM((n_chunks, _CHUNK), jnp.int32),
            pltpu.VMEM((n_chunks, _CHUNK), jnp.int32),
            pltpu.VMEM((rows_per_w, D), jnp.float32),
            pltpu.VMEM((rows_per_w, D), jnp.float32),
            pltpu.SemaphoreType.DMA,
        ],
        compiler_params=pltpu.CompilerParams(
            use_tc_tiling_on_sc=False, needs_layout_passes=False),
    )
    def deepwalk_lookup(srcs_hbm, dsts_hbm, table_hbm, out_s, out_d,
                        idx_s, idx_d, rows_s, rows_d, sem):
        wid = lax.axis_index("s") * _NUM_CORES + lax.axis_index("c")
        crow = wid * n_chunks
        base = wid * rows_per_w
        pltpu.sync_copy(srcs_hbm.at[pl.ds(crow, n_chunks)], idx_s)
        pltpu.sync_copy(dsts_hbm.at[pl.ds(crow, n_chunks)], idx_d)
        copies = []
        for c in range(n_chunks):
            for r in range(_CHUNK // 16):
                vs = idx_s[c, pl.ds(r * 16, 16)]
                vd = idx_d[c, pl.ds(r * 16, 16)]
                o = c * _CHUNK + r * 16
                copies.append(pltpu.async_copy(
                    table_hbm.at[vs], rows_s.at[pl.ds(o, 16)], sem))
                copies.append(pltpu.async_copy(
                    table_hbm.at[vd], rows_d.at[pl.ds(o, 16)], sem))
        for cp in copies:
            cp.wait()
        pltpu.sync_copy(rows_s, out_s.at[pl.ds(base, rows_per_w)])
        pltpu.sync_copy(rows_d, out_d.at[pl.ds(base, rows_per_w)])

    return deepwalk_lookup(srcs2, dsts2, table)


# speedup vs baseline: 1.5783x; 1.1702x over previous
import functools

import jax
import jax.numpy as jnp
from jax import lax
from jax.experimental import pallas as pl
from jax.experimental.pallas import tpu as pltpu
from jax.experimental.pallas import tpu_sc as plsc

_NUM_CORES = 2
_NUM_SUBCORES = 16
_NW = _NUM_CORES * _NUM_SUBCORES
_LANES = 16
_DIRECT = 160


def kernel(srcs, dsts, table):
    B = srcs.shape[0]
    D = table.shape[1]
    rows_per_w = B // _NW
    n_vregs = rows_per_w // _LANES
    stream_rows = rows_per_w - _DIRECT
    sv = stream_rows // _LANES

    mesh = plsc.VectorSubcoreMesh(
        core_axis_name="c", subcore_axis_name="s",
        num_cores=_NUM_CORES, num_subcores=_NUM_SUBCORES)

    @functools.partial(
        pl.kernel,
        out_type=(jax.ShapeDtypeStruct((B, D), jnp.float32),
                  jax.ShapeDtypeStruct((B, D), jnp.float32)),
        mesh=mesh,
        scratch_types=[
            pltpu.VMEM((rows_per_w,), jnp.int32),
            pltpu.VMEM((rows_per_w,), jnp.int32),
            pltpu.VMEM((stream_rows, D), jnp.float32),
            pltpu.VMEM((stream_rows, D), jnp.float32),
            pltpu.SemaphoreType.DMA,
            pltpu.SemaphoreType.DMA,
        ],
        compiler_params=pltpu.CompilerParams(needs_layout_passes=False),
    )
    def deepwalk_lookup(srcs_hbm, dsts_hbm, table_hbm, out_s, out_d,
                        idx_s, idx_d, rows_s, rows_d, sem_s, sem_d):
        wid = lax.axis_index("s") * _NUM_CORES + lax.axis_index("c")
        base = wid * rows_per_w
        pltpu.sync_copy(srcs_hbm.at[pl.ds(base, rows_per_w)], idx_s)
        pltpu.sync_copy(dsts_hbm.at[pl.ds(base, rows_per_w)], idx_d)

        def dma_body(idx_ref, out_ref):
            def body(rv, carry):
                v = idx_ref[pl.ds((sv + rv) * _LANES, _LANES)]
                for l in range(_LANES):
                    i = v[l]
                    pltpu.async_copy(
                        table_hbm.at[i],
                        out_ref.at[base + (sv + rv) * _LANES + l], sem_d)
                return carry
            lax.fori_loop(0, n_vregs - sv, body, 0)

        def stream_body(idx_ref, rows_ref):
            def body(rv, carry):
                copies = []
                v = idx_ref[pl.ds(rv * _LANES, _LANES)]
                for l in range(_LANES):
                    i = v[l]
                    copies.append(pltpu.async_copy(
                        table_hbm.at[i],
                        rows_ref.at[rv * _LANES + l], sem_s))
                for cp in copies:
                    cp.wait()
                return carry
            lax.fori_loop(0, sv, body, 0)

        dma_body(idx_s, out_s)
        dma_body(idx_d, out_d)
        stream_body(idx_s, rows_s)
        pltpu.sync_copy(rows_s, out_s.at[pl.ds(base, stream_rows)])
        stream_body(idx_d, rows_d)
        pltpu.sync_copy(rows_d, out_d.at[pl.ds(base, stream_rows)])

        def drain_body(out_ref):
            def body(rv, carry):
                for l in range(_LANES):
                    pltpu.make_async_copy(
                        table_hbm.at[0],
                        out_ref.at[base + (sv + rv) * _LANES + l],
                        sem_d).wait()
                return carry
            lax.fori_loop(0, n_vregs - sv, body, 0)

        drain_body(out_s)
        drain_body(out_d)

    return deepwalk_lookup(srcs, dsts, table)


# speedup vs baseline: 1.6624x; 1.0533x over previous
import functools

import jax
import jax.numpy as jnp
from jax import lax
from jax.experimental import pallas as pl
from jax.experimental.pallas import tpu as pltpu
from jax.experimental.pallas import tpu_sc as plsc

_NUM_CORES = 2
_NUM_SUBCORES = 16
_NW = _NUM_CORES * _NUM_SUBCORES
_LANES = 16
_DIRECT = 160


def kernel(srcs, dsts, table):
    B = srcs.shape[0]
    D = table.shape[1]
    rows_per_w = B // _NW
    n_vregs = rows_per_w // _LANES
    stream_rows = rows_per_w - _DIRECT
    sv = stream_rows // _LANES

    mesh = plsc.VectorSubcoreMesh(
        core_axis_name="c", subcore_axis_name="s",
        num_cores=_NUM_CORES, num_subcores=_NUM_SUBCORES)

    @functools.partial(
        pl.kernel,
        out_type=(jax.ShapeDtypeStruct((B, D), jnp.float32),
                  jax.ShapeDtypeStruct((B, D), jnp.float32)),
        mesh=mesh,
        scratch_types=[
            pltpu.VMEM((rows_per_w,), jnp.int32),
            pltpu.VMEM((rows_per_w,), jnp.int32),
            pltpu.VMEM((stream_rows, D), jnp.float32),
            pltpu.VMEM((stream_rows, D), jnp.float32),
            pltpu.SemaphoreType.DMA,
            pltpu.SemaphoreType.DMA,
        ],
        compiler_params=pltpu.CompilerParams(needs_layout_passes=False),
    )
    def deepwalk_lookup(srcs_hbm, dsts_hbm, table_hbm, out_s, out_d,
                        idx_s, idx_d, rows_s, rows_d, sem_s, sem_d):
        wid = lax.axis_index("s") * _NUM_CORES + lax.axis_index("c")
        base = wid * rows_per_w
        pltpu.sync_copy(srcs_hbm.at[pl.ds(base, rows_per_w)], idx_s)
        pltpu.sync_copy(dsts_hbm.at[pl.ds(base, rows_per_w)], idx_d)

        n_dma_v = n_vregs - sv

        def issue_stream_vreg(idx_ref, rows_ref, rv):
            copies = []
            v = idx_ref[pl.ds(rv * _LANES, _LANES)]
            for l in range(_LANES):
                copies.append(pltpu.async_copy(
                    table_hbm.at[v[l]], rows_ref.at[rv * _LANES + l],
                    sem_s))
            for cp in copies:
                cp.wait()

        def run_list(idx_ref, rows_ref, out_ref):
            def round_body(rv, carry):
                dv = sv + rv
                v = idx_ref[pl.ds(dv * _LANES, _LANES)]
                for l in range(_LANES):
                    pltpu.async_copy(
                        table_hbm.at[v[l]],
                        out_ref.at[base + dv * _LANES + l], sem_d)
                issue_stream_vreg(idx_ref, rows_ref, 2 * rv)
                issue_stream_vreg(idx_ref, rows_ref, 2 * rv + 1)
                return carry

            lax.fori_loop(0, n_dma_v, round_body, 0)

            def tail_body(rv, carry):
                issue_stream_vreg(idx_ref, rows_ref, 2 * n_dma_v + rv)
                return carry

            lax.fori_loop(0, sv - 2 * n_dma_v, tail_body, 0)
            pltpu.sync_copy(rows_ref, out_ref.at[pl.ds(base, stream_rows)])

        run_list(idx_s, rows_s, out_s)
        run_list(idx_d, rows_d, out_d)

        def drain_body(out_ref):
            def body(rv, carry):
                for l in range(_LANES):
                    pltpu.make_async_copy(
                        table_hbm.at[0],
                        out_ref.at[base + (sv + rv) * _LANES + l],
                        sem_d).wait()
                return carry
            lax.fori_loop(0, n_vregs - sv, body, 0)

        drain_body(out_s)
        drain_body(out_d)

    return deepwalk_lookup(srcs, dsts, table)


# speedup vs baseline: 2.2621x; 1.3608x over previous
import functools

import jax
import jax.numpy as jnp
from jax import lax
from jax.experimental import pallas as pl
from jax.experimental.pallas import tpu as pltpu
from jax.experimental.pallas import tpu_sc as plsc

_NUM_CORES = 2
_NUM_SUBCORES = 16
_NW = _NUM_CORES * _NUM_SUBCORES
_CHUNK = 128
_LANES = 16


def kernel(srcs, dsts, table):
    B = srcs.shape[0]
    D = table.shape[1]
    rows_per_w = B // _NW
    n_chunks = rows_per_w // _CHUNK

    srcs2 = srcs.reshape(B // _CHUNK, _CHUNK)
    dsts2 = dsts.reshape(B // _CHUNK, _CHUNK)

    mesh = plsc.VectorSubcoreMesh(
        core_axis_name="c", subcore_axis_name="s",
        num_cores=_NUM_CORES, num_subcores=_NUM_SUBCORES)

    @functools.partial(
        pl.kernel,
        out_type=(jax.ShapeDtypeStruct((B, D), jnp.float32),
                  jax.ShapeDtypeStruct((B, D), jnp.float32)),
        mesh=mesh,
        scratch_types=[
            pltpu.VMEM((n_chunks, _CHUNK), jnp.int32),
            pltpu.VMEM((n_chunks, _CHUNK), jnp.int32),
            pltpu.VMEM((_CHUNK, D), jnp.float32),
            pltpu.VMEM((_CHUNK, D), jnp.float32),
            pltpu.SemaphoreType.DMA,
        ],
    )
    def deepwalk_lookup(srcs_hbm, dsts_hbm, table_hbm, out_s, out_d,
                        idx_s, idx_d, rows_s, rows_d, sem):
        wid = lax.axis_index("s") * _NUM_CORES + lax.axis_index("c")
        crow = wid * n_chunks
        pltpu.sync_copy(srcs_hbm.at[pl.ds(crow, n_chunks)], idx_s)
        pltpu.sync_copy(dsts_hbm.at[pl.ds(crow, n_chunks)], idx_d)
        base = wid * rows_per_w

        def run_list(idx_ref, rows_ref, out_ref):
            def chunk_body(c, carry):
                copies = []
                for r in range(_CHUNK // _LANES):
                    v = idx_ref[c, pl.ds(r * _LANES, _LANES)]
                    for l in range(_LANES):
                        i = v[l]
                        copies.append(pltpu.async_copy(
                            table_hbm.at[i], rows_ref.at[r * _LANES + l],
                            sem))
                for cp in copies:
                    cp.wait()
                pltpu.sync_copy(
                    rows_ref, out_ref.at[pl.ds(base + c * _CHUNK, _CHUNK)])
                return carry

            lax.fori_loop(0, n_chunks, chunk_body, 0)

        run_list(idx_s, rows_s, out_s)
        run_list(idx_d, rows_d, out_d)

    return deepwalk_lookup(srcs2, dsts2, table)
